# baseline (speedup 1.0000x reference)
import functools

import jax
import jax.numpy as jnp
from jax import lax
from jax.experimental import pallas as pl
from jax.experimental.pallas import tpu as pltpu
from jax.experimental.pallas import tpu_sc as plsc

_N_ROWS = 10000
_N_UNITS = 2
_ROW_BLK = 1000
_LANES = 16


def _lut_body(f_ref, k_ref, b_ref, o_ref):
    absf = jnp.abs(f_ref[0])
    ones = jnp.ones((8, 128), jnp.float32)
    rs8 = lax.dot_general(ones, absf, (((1,), (1,)), ((), ())),
                          precision=lax.Precision.HIGHEST)
    o_ref[...] = jax.nn.sigmoid(rs8[:_N_UNITS] * k_ref[...] + b_ref[...])


def _build_lut(feature, kern, bias):
    return pl.pallas_call(
        _lut_body,
        out_shape=jax.ShapeDtypeStruct((_N_UNITS, _N_ROWS), jnp.float32),
    )(feature, kern.reshape(_N_UNITS, 1), bias.reshape(_N_UNITS, 1))


def _gather_lut(ppi_t, lut, n_edges):
    info = plsc.get_sparse_core_info()
    nc, ns = info.num_cores, info.num_subcores
    nw = nc * ns
    chunk_e = n_edges // nw

    mesh = plsc.VectorSubcoreMesh(core_axis_name="c", subcore_axis_name="s")

    @functools.partial(
        pl.kernel,
        mesh=mesh,
        out_type=jax.ShapeDtypeStruct((_N_UNITS, n_edges), jnp.float32),
        scratch_types=[
            pltpu.VMEM((chunk_e,), jnp.int32),
            pltpu.VMEM((chunk_e,), jnp.int32),
            pltpu.VMEM((_N_ROWS,), jnp.float32),
            pltpu.VMEM((_N_ROWS,), jnp.float32),
            pltpu.VMEM((chunk_e,), jnp.float32),
            pltpu.VMEM((chunk_e,), jnp.float32),
            pltpu.SemaphoreType.DMA,
            pltpu.SemaphoreType.DMA,
            pltpu.SemaphoreType.DMA,
            pltpu.SemaphoreType.DMA,
            pltpu.SemaphoreType.DMA,
            pltpu.SemaphoreType.DMA,
        ],
        compiler_params=pltpu.CompilerParams(
            use_tc_tiling_on_sc=False,
            needs_layout_passes=False,
        ),
    )
    def gather_k(ppi_hbm, lut_hbm, out_hbm,
                 idx0_v, idx1_v, lut0_v, lut1_v, out0_v, out1_v,
                 s0, s1, s2, s3, s4, s5):
        wid = lax.axis_index("s") * nc + lax.axis_index("c")
        base = wid * chunk_e
        n_vec = chunk_e // _LANES
        half_v = n_vec // 2
        half_e = half_v * _LANES
        rest_e = chunk_e - half_e
        cl0 = pltpu.async_copy(lut_hbm.at[0, :], lut0_v, s0)
        ci0a = pltpu.async_copy(
            ppi_hbm.at[0, pl.ds(base, half_e)], idx0_v.at[pl.ds(0, half_e)], s1)
        ci0b = pltpu.async_copy(
            ppi_hbm.at[0, pl.ds(base + half_e, rest_e)],
            idx0_v.at[pl.ds(half_e, rest_e)], s2)
        cl1 = pltpu.async_copy(lut_hbm.at[1, :], lut1_v, s3)
        ci1a = pltpu.async_copy(
            ppi_hbm.at[1, pl.ds(base, half_e)], idx1_v.at[pl.ds(0, half_e)], s4)
        ci1b = pltpu.async_copy(
            ppi_hbm.at[1, pl.ds(base + half_e, rest_e)],
            idx1_v.at[pl.ds(half_e, rest_e)], s5)
        cl0.wait()
        ci0a.wait()

        @plsc.parallel_loop(0, half_v, unroll=8)
        def body0a(i):
            sl = pl.ds(i * _LANES, _LANES)
            out0_v[sl] = plsc.load_gather(lut0_v, [idx0_v[sl]])

        ci0b.wait()

        @plsc.parallel_loop(half_v, n_vec, unroll=8)
        def body0b(i):
            sl = pl.ds(i * _LANES, _LANES)
            out0_v[sl] = plsc.load_gather(lut0_v, [idx0_v[sl]])

        co0 = pltpu.async_copy(out0_v, out_hbm.at[0, pl.ds(base, chunk_e)], s1)
        cl1.wait()
        ci1a.wait()

        @plsc.parallel_loop(0, half_v, unroll=8)
        def body1a(i):
            sl = pl.ds(i * _LANES, _LANES)
            out1_v[sl] = plsc.load_gather(lut1_v, [idx1_v[sl]])

        ci1b.wait()

        @plsc.parallel_loop(half_v, n_vec, unroll=8)
        def body1b(i):
            sl = pl.ds(i * _LANES, _LANES)
            out1_v[sl] = plsc.load_gather(lut1_v, [idx1_v[sl]])

        co1 = pltpu.async_copy(out1_v, out_hbm.at[1, pl.ds(base, chunk_e)], s2)
        co0.wait()
        co1.wait()

    return gather_k(ppi_t, lut)


def kernel(feature, ppi, kernel, bias):
    n_edges = ppi.shape[0]
    lut = _build_lut(feature, kernel, bias)
    out_t = _gather_lut(ppi.T, lut, n_edges)
    return out_t.T[None]


# speedup vs baseline: 1.0022x; 1.0022x over previous
import functools

import jax
import jax.numpy as jnp
from jax import lax
from jax.experimental import pallas as pl
from jax.experimental.pallas import tpu as pltpu
from jax.experimental.pallas import tpu_sc as plsc

_N_ROWS = 10000
_N_UNITS = 2
_LANES = 16


def _lut_body(f_ref, k_ref, b_ref, o_ref):
    absf = jnp.abs(f_ref[0])
    ones = jnp.ones((8, 128), jnp.float32)
    rs8 = lax.dot_general(ones, absf, (((1,), (1,)), ((), ())),
                          precision=lax.Precision.HIGHEST)
    o_ref[...] = jax.nn.sigmoid(rs8[:_N_UNITS] * k_ref[...] + b_ref[...])


def _build_lut(feature, kern, bias):
    return pl.pallas_call(
        _lut_body,
        out_shape=jax.ShapeDtypeStruct((_N_UNITS, _N_ROWS), jnp.float32),
    )(feature, kern.reshape(_N_UNITS, 1), bias.reshape(_N_UNITS, 1))


def _gather_lut(ppi_t, lut, n_edges):
    info = plsc.get_sparse_core_info()
    nc, ns = info.num_cores, info.num_subcores
    nw = nc * ns
    chunk_e = n_edges // nw

    mesh = plsc.VectorSubcoreMesh(core_axis_name="c", subcore_axis_name="s")

    @functools.partial(
        pl.kernel,
        mesh=mesh,
        out_type=jax.ShapeDtypeStruct((_N_UNITS, n_edges), jnp.float32),
        scratch_types=[
            pltpu.VMEM((chunk_e,), jnp.int32),
            pltpu.VMEM((chunk_e,), jnp.int32),
            pltpu.VMEM((_N_ROWS,), jnp.float32),
            pltpu.VMEM((_N_ROWS,), jnp.float32),
            pltpu.VMEM((chunk_e,), jnp.float32),
            pltpu.VMEM((chunk_e,), jnp.float32),
            pltpu.SemaphoreType.DMA,
            pltpu.SemaphoreType.DMA,
            pltpu.SemaphoreType.DMA,
            pltpu.SemaphoreType.DMA,
            pltpu.SemaphoreType.DMA,
            pltpu.SemaphoreType.DMA,
        ],
        compiler_params=pltpu.CompilerParams(
            use_tc_tiling_on_sc=False,
            needs_layout_passes=False,
        ),
    )
    def gather_k(ppi_hbm, lut_hbm, out_hbm,
                 idx0_v, idx1_v, lut0_v, lut1_v, out0_v, out1_v,
                 s0, s1, s2, s3, s4, s5):
        wid = lax.axis_index("s") * nc + lax.axis_index("c")
        base = wid * chunk_e
        n_vec = chunk_e // _LANES
        half_v = n_vec // 2
        half_e = half_v * _LANES
        rest_e = chunk_e - half_e
        cl0 = pltpu.async_copy(lut_hbm.at[0, :], lut0_v, s0)
        ci0a = pltpu.async_copy(
            ppi_hbm.at[0, pl.ds(base, half_e)], idx0_v.at[pl.ds(0, half_e)], s1)
        ci0b = pltpu.async_copy(
            ppi_hbm.at[0, pl.ds(base + half_e, rest_e)],
            idx0_v.at[pl.ds(half_e, rest_e)], s2)
        cl1 = pltpu.async_copy(lut_hbm.at[1, :], lut1_v, s3)
        ci1a = pltpu.async_copy(
            ppi_hbm.at[1, pl.ds(base, half_e)], idx1_v.at[pl.ds(0, half_e)], s4)
        ci1b = pltpu.async_copy(
            ppi_hbm.at[1, pl.ds(base + half_e, rest_e)],
            idx1_v.at[pl.ds(half_e, rest_e)], s5)
        cl0.wait()
        ci0a.wait()

        @plsc.parallel_loop(0, half_v, unroll=8)
        def body0a(i):
            sl = pl.ds(i * _LANES, _LANES)
            out0_v[sl] = plsc.load_gather(lut0_v, [idx0_v[sl]])

        ci0b.wait()

        @plsc.parallel_loop(half_v, n_vec, unroll=8)
        def body0b(i):
            sl = pl.ds(i * _LANES, _LANES)
            out0_v[sl] = plsc.load_gather(lut0_v, [idx0_v[sl]])

        co0 = pltpu.async_copy(out0_v, out_hbm.at[0, pl.ds(base, chunk_e)], s1)
        cl1.wait()
        ci1a.wait()

        @plsc.parallel_loop(0, half_v, unroll=8)
        def body1a(i):
            sl = pl.ds(i * _LANES, _LANES)
            out1_v[sl] = plsc.load_gather(lut1_v, [idx1_v[sl]])

        ci1b.wait()

        @plsc.parallel_loop(half_v, n_vec, unroll=8)
        def body1b(i):
            sl = pl.ds(i * _LANES, _LANES)
            out1_v[sl] = plsc.load_gather(lut1_v, [idx1_v[sl]])

        co1 = pltpu.async_copy(out1_v, out_hbm.at[1, pl.ds(base, chunk_e)], s2)
        co0.wait()
        co1.wait()

    return gather_k(ppi_t, lut)


def kernel(feature, ppi, kernel, bias):
    n_edges = ppi.shape[0]
    lut = _build_lut(feature, kernel, bias)
    out_t = _gather_lut(ppi.T, lut, n_edges)
    return out_t.T[None]
